# baseline (speedup 1.0000x reference)
import jax
import jax.numpy as jnp
from jax import lax
from jax.experimental import pallas as pl
from jax.experimental.pallas import tpu as pltpu

D_MODEL = 2048
D_FFN = 8192
K = 128
CHUNK = 512
NCHUNK = D_FFN // CHUNK
MM_CHUNK = 256
MM_N = D_FFN // MM_CHUNK


def _gu_kernel(x_ref, wg_ref, wu_ref, g_ref, u_ref):
    x = x_ref[...]
    g_ref[...] = jnp.dot(x, wg_ref[...], preferred_element_type=jnp.float32)
    u_ref[...] = jnp.dot(x, wu_ref[...], preferred_element_type=jnp.float32)


def _select_kernel(g_ref, u_ref, z_ref):
    g = g_ref[...]
    m = g.shape[0]
    b = lax.bitcast_convert_type(g, jnp.int32)
    keys = b ^ ((b >> 31) & jnp.int32(0x7FFFFFFF))
    cnt_pos = jnp.sum((keys >= 0).astype(jnp.int32), axis=1, keepdims=True)
    t0 = jnp.where(cnt_pos >= K, jnp.int32(0), jnp.int32(-(2**31)))

    def body(i, t):
        cand = t | (jnp.int32(1) << (jnp.int32(30) - i))
        cnt = jnp.sum((keys >= cand).astype(jnp.int32), axis=1, keepdims=True)
        return jnp.where(cnt >= K, cand, t)

    t = lax.fori_loop(0, 31, body, t0)
    n_gt = jnp.sum((keys > t).astype(jnp.int32), axis=1, keepdims=True)
    need = (K - n_gt).astype(jnp.float32)

    u = u_ref[...]
    zfull = g * (1.0 / (1.0 + jnp.exp(-g))) * u

    ra = lax.broadcasted_iota(jnp.int32, (CHUNK, CHUNK), 0)
    rc = lax.broadcasted_iota(jnp.int32, (CHUNK, CHUNK), 1)
    tri = (ra <= rc).astype(jnp.float32)
    carry = jnp.zeros((m, 1), jnp.float32)
    for c in range(NCHUNK):
        kc = lax.slice(keys, (0, c * CHUNK), (m, (c + 1) * CHUNK))
        eq = (kc == t).astype(jnp.float32)
        cum = jnp.dot(eq, tri, preferred_element_type=jnp.float32) + carry
        sel = (kc > t) | ((kc == t) & (cum <= need))
        zc = lax.slice(zfull, (0, c * CHUNK), (m, (c + 1) * CHUNK))
        z_ref[:, c * CHUNK:(c + 1) * CHUNK] = jnp.where(sel, zc, 0.0)
        carry = carry + jnp.sum(eq, axis=1, keepdims=True)


def _down_kernel(z_ref, wd_ref, o_ref, acc_ref):
    @pl.when(pl.program_id(0) == 0)
    def _init():
        acc_ref[...] = jnp.zeros_like(acc_ref)

    acc_ref[...] += jnp.dot(z_ref[...], wd_ref[...],
                            preferred_element_type=jnp.float32)

    @pl.when(pl.program_id(0) == pl.num_programs(0) - 1)
    def _emit():
        o_ref[...] = acc_ref[...]


def kernel(x, w_gate, w_up, w_down):
    orig_shape = x.shape
    xf = x.reshape(-1, orig_shape[-1])
    m = xf.shape[0]

    g, u = pl.pallas_call(
        _gu_kernel,
        grid=(MM_N,),
        in_specs=[
            pl.BlockSpec((m, D_MODEL), lambda c: (0, 0)),
            pl.BlockSpec((D_MODEL, MM_CHUNK), lambda c: (0, c)),
            pl.BlockSpec((D_MODEL, MM_CHUNK), lambda c: (0, c)),
        ],
        out_specs=[pl.BlockSpec((m, MM_CHUNK), lambda c: (0, c))] * 2,
        out_shape=[jax.ShapeDtypeStruct((m, D_FFN), jnp.float32)] * 2,
    )(xf, w_gate, w_up)

    z = pl.pallas_call(
        _select_kernel,
        out_shape=jax.ShapeDtypeStruct((m, D_FFN), jnp.float32),
    )(g, u)

    out = pl.pallas_call(
        _down_kernel,
        grid=(MM_N,),
        in_specs=[
            pl.BlockSpec((m, MM_CHUNK), lambda c: (0, c)),
            pl.BlockSpec((MM_CHUNK, D_MODEL), lambda c: (c, 0)),
        ],
        out_specs=pl.BlockSpec((m, D_MODEL), lambda c: (0, 0)),
        out_shape=jax.ShapeDtypeStruct((m, D_MODEL), jnp.float32),
        scratch_shapes=[pltpu.VMEM((m, D_MODEL), jnp.float32)],
    )(z, w_down)

    return out.reshape(orig_shape)


# speedup vs baseline: 1.2145x; 1.2145x over previous
import jax
import jax.numpy as jnp
from jax import lax
from jax.experimental import pallas as pl
from jax.experimental.pallas import tpu as pltpu

D_MODEL = 2048
D_FFN = 8192
K = 128
CHUNK = 512
NCHUNK = D_FFN // CHUNK


def _keys_of(g):
    b = lax.bitcast_convert_type(g, jnp.int32)
    return b ^ ((b >> 31) & jnp.int32(0x7FFFFFFF))


def _fused_kernel(x_ref, wg_ref, wu_ref, wd_ref, o_ref,
                  g_s, u_s, tn_s, carry_s, acc_ref):
    c = pl.program_id(0)
    m = x_ref.shape[0]

    @pl.when(c < NCHUNK)
    def _gu():
        x = x_ref[...]
        sl = pl.ds(c * CHUNK, CHUNK)
        g_s[:, sl] = jnp.dot(x, wg_ref[...], preferred_element_type=jnp.float32)
        u_s[:, sl] = jnp.dot(x, wu_ref[...], preferred_element_type=jnp.float32)

    @pl.when(c == NCHUNK)
    def _threshold():
        keys = _keys_of(g_s[...])
        cnt_pos = jnp.sum((keys >= 0).astype(jnp.int32), axis=1,
                          keepdims=True)
        t0 = jnp.where(cnt_pos >= K, jnp.int32(0), jnp.int32(-(2**31)))

        def body(i, t):
            cand = t | (jnp.int32(1) << (jnp.int32(30) - i))
            cnt = jnp.sum((keys >= cand).astype(jnp.int32), axis=1,
                          keepdims=True)
            return jnp.where(cnt >= K, cand, t)

        t = lax.fori_loop(0, 31, body, t0)
        n_gt = jnp.sum((keys > t).astype(jnp.int32), axis=1, keepdims=True)
        col = lax.broadcasted_iota(jnp.int32, (m, 128), 1)
        tn_s[...] = jnp.where(col < 64, t, K - n_gt)
        carry_s[...] = jnp.zeros_like(carry_s)

    @pl.when(c > NCHUNK)
    def _down():
        cc = c - (NCHUNK + 1)
        g = g_s[:, pl.ds(cc * CHUNK, CHUNK)]
        u = u_s[:, pl.ds(cc * CHUNK, CHUNK)]
        kc = _keys_of(g)
        tn = tn_s[...]
        t = lax.slice(tn, (0, 0), (m, 1))
        need = lax.slice(tn, (0, 64), (m, 65)).astype(jnp.float32)
        ra = lax.broadcasted_iota(jnp.int32, (CHUNK, CHUNK), 0)
        rc = lax.broadcasted_iota(jnp.int32, (CHUNK, CHUNK), 1)
        tri = (ra <= rc).astype(jnp.float32)
        eq = (kc == t).astype(jnp.float32)
        carry = lax.slice(carry_s[...], (0, 0), (m, 1))
        cum = jnp.dot(eq, tri, preferred_element_type=jnp.float32) + carry
        sel = (kc > t) | ((kc == t) & (cum <= need))
        z = jnp.where(sel, g * (1.0 / (1.0 + jnp.exp(-g))) * u, 0.0)
        carry_s[...] = jnp.broadcast_to(
            carry + jnp.sum(eq, axis=1, keepdims=True), carry_s.shape)
        zd = jnp.dot(z, wd_ref[...], preferred_element_type=jnp.float32)

        @pl.when(cc == 0)
        def _first():
            acc_ref[...] = zd

        @pl.when(cc > 0)
        def _rest():
            acc_ref[...] += zd

        @pl.when(cc == NCHUNK - 1)
        def _emit():
            o_ref[...] = acc_ref[...]


def kernel(x, w_gate, w_up, w_down):
    orig_shape = x.shape
    xf = x.reshape(-1, orig_shape[-1])
    m = xf.shape[0]

    out = pl.pallas_call(
        _fused_kernel,
        grid=(2 * NCHUNK + 1,),
        in_specs=[
            pl.BlockSpec((m, D_MODEL), lambda c: (0, 0)),
            pl.BlockSpec((D_MODEL, CHUNK),
                         lambda c: (0, jnp.minimum(c, NCHUNK - 1))),
            pl.BlockSpec((D_MODEL, CHUNK),
                         lambda c: (0, jnp.minimum(c, NCHUNK - 1))),
            pl.BlockSpec((CHUNK, D_MODEL),
                         lambda c: (jnp.clip(c - NCHUNK - 1, 0, NCHUNK - 1),
                                    0)),
        ],
        out_specs=pl.BlockSpec((m, D_MODEL), lambda c: (0, 0)),
        out_shape=jax.ShapeDtypeStruct((m, D_MODEL), jnp.float32),
        scratch_shapes=[
            pltpu.VMEM((m, D_FFN), jnp.float32),
            pltpu.VMEM((m, D_FFN), jnp.float32),
            pltpu.VMEM((m, 128), jnp.int32),
            pltpu.VMEM((m, 128), jnp.float32),
            pltpu.VMEM((m, D_MODEL), jnp.float32),
        ],
    )(xf, w_gate, w_up, w_down)

    return out.reshape(orig_shape)


# speedup vs baseline: 1.3975x; 1.1507x over previous
import jax
import jax.numpy as jnp
from jax import lax
from jax.experimental import pallas as pl
from jax.experimental.pallas import tpu as pltpu

D_MODEL = 2048
D_FFN = 8192
K = 128
CHUNK = 512
NCHUNK = D_FFN // CHUNK


def _keys_of(g):
    b = lax.bitcast_convert_type(g, jnp.int32)
    return b ^ ((b >> 31) & jnp.int32(0x7FFFFFFF))


NBUF = 6


def _fused_kernel(x_ref, wg_ref, wu_ref, wd_ref, o_ref,
                  g_s, u_s, tn_s, carry_s, acc_ref, wd_buf, wd_sem):
    c = pl.program_id(0)
    m = x_ref.shape[0]

    def _start_wd(chunk, slot):
        pltpu.make_async_copy(
            wd_ref.at[pl.ds(chunk * CHUNK, CHUNK), :],
            wd_buf.at[slot], wd_sem.at[slot]).start()

    @pl.when(c < NCHUNK)
    def _gu():
        x = x_ref[...]
        sl = pl.ds(c * CHUNK, CHUNK)
        g_s[:, sl] = jnp.dot(x, wg_ref[...], preferred_element_type=jnp.float32)
        u_s[:, sl] = jnp.dot(x, wu_ref[...], preferred_element_type=jnp.float32)

    @pl.when(c == NCHUNK)
    def _threshold():
        for k in range(NBUF):
            _start_wd(k, k)
        keys = _keys_of(g_s[...])
        cnt_pos = jnp.sum((keys >= 0).astype(jnp.int32), axis=1,
                          keepdims=True)
        t0 = jnp.where(cnt_pos >= K, jnp.int32(0), jnp.int32(-(2**31)))

        def body(i, t):
            cand = t | (jnp.int32(1) << (jnp.int32(30) - i))
            cnt = jnp.sum((keys >= cand).astype(jnp.int32), axis=1,
                          keepdims=True)
            return jnp.where(cnt >= K, cand, t)

        t = lax.fori_loop(0, 31, body, t0)
        n_gt = jnp.sum((keys > t).astype(jnp.int32), axis=1, keepdims=True)
        col = lax.broadcasted_iota(jnp.int32, (m, 128), 1)
        tn_s[...] = jnp.where(col < 64, t, K - n_gt)
        carry_s[...] = jnp.zeros_like(carry_s)

    @pl.when(c > NCHUNK)
    def _down():
        cc = c - (NCHUNK + 1)
        g = g_s[:, pl.ds(cc * CHUNK, CHUNK)]
        u = u_s[:, pl.ds(cc * CHUNK, CHUNK)]
        kc = _keys_of(g)
        tn = tn_s[...]
        t = lax.slice(tn, (0, 0), (m, 1))
        need = lax.slice(tn, (0, 64), (m, 65)).astype(jnp.float32)
        ra = lax.broadcasted_iota(jnp.int32, (CHUNK, CHUNK), 0)
        rc = lax.broadcasted_iota(jnp.int32, (CHUNK, CHUNK), 1)
        tri = (ra <= rc).astype(jnp.float32)
        eq = (kc == t).astype(jnp.float32)
        carry = lax.slice(carry_s[...], (0, 0), (m, 1))
        cum = jnp.dot(eq, tri, preferred_element_type=jnp.float32) + carry
        sel = (kc > t) | ((kc == t) & (cum <= need))
        z = jnp.where(sel, g * (1.0 / (1.0 + jnp.exp(-g))) * u, 0.0)
        carry_s[...] = jnp.broadcast_to(
            carry + jnp.sum(eq, axis=1, keepdims=True), carry_s.shape)
        slot = lax.rem(cc, NBUF)
        pltpu.make_async_copy(
            wd_ref.at[pl.ds(cc * CHUNK, CHUNK), :],
            wd_buf.at[slot], wd_sem.at[slot]).wait()
        zd = jnp.dot(z, wd_buf[slot], preferred_element_type=jnp.float32)

        @pl.when(cc + NBUF < NCHUNK)
        def _refill():
            _start_wd(cc + NBUF, slot)

        @pl.when(cc == 0)
        def _first():
            acc_ref[...] = zd

        @pl.when(cc > 0)
        def _rest():
            acc_ref[...] += zd

        @pl.when(cc == NCHUNK - 1)
        def _emit():
            o_ref[...] = acc_ref[...]


def kernel(x, w_gate, w_up, w_down):
    orig_shape = x.shape
    xf = x.reshape(-1, orig_shape[-1])
    m = xf.shape[0]

    out = pl.pallas_call(
        _fused_kernel,
        grid=(2 * NCHUNK + 1,),
        in_specs=[
            pl.BlockSpec((m, D_MODEL), lambda c: (0, 0)),
            pl.BlockSpec((D_MODEL, CHUNK),
                         lambda c: (0, jnp.minimum(c, NCHUNK - 1))),
            pl.BlockSpec((D_MODEL, CHUNK),
                         lambda c: (0, jnp.minimum(c, NCHUNK - 1))),
            pl.BlockSpec(memory_space=pl.ANY),
        ],
        out_specs=pl.BlockSpec((m, D_MODEL), lambda c: (0, 0)),
        out_shape=jax.ShapeDtypeStruct((m, D_MODEL), jnp.float32),
        scratch_shapes=[
            pltpu.VMEM((m, D_FFN), jnp.float32),
            pltpu.VMEM((m, D_FFN), jnp.float32),
            pltpu.VMEM((m, 128), jnp.int32),
            pltpu.VMEM((m, 128), jnp.float32),
            pltpu.VMEM((m, D_MODEL), jnp.float32),
            pltpu.VMEM((NBUF, CHUNK, D_MODEL), jnp.float32),
            pltpu.SemaphoreType.DMA((NBUF,)),
        ],
    )(xf, w_gate, w_up, w_down)

    return out.reshape(orig_shape)


# speedup vs baseline: 1.4334x; 1.0257x over previous
import jax
import jax.numpy as jnp
from jax import lax
from jax.experimental import pallas as pl
from jax.experimental.pallas import tpu as pltpu

D_MODEL = 2048
D_FFN = 8192
K = 128
CHUNK = 512
NCHUNK = D_FFN // CHUNK


def _keys_of(g):
    b = lax.bitcast_convert_type(g, jnp.int32)
    return b ^ ((b >> 31) & jnp.int32(0x7FFFFFFF))


NBUF = 8


def _fused_kernel(x_ref, wg_ref, wu_ref, wd_ref, o_ref,
                  g_s, u_s, tn_s, carry_s, acc_ref, wd_buf, wd_sem):
    c = pl.program_id(0)
    m = x_ref.shape[0]

    def _start_wd(chunk, slot):
        pltpu.make_async_copy(
            wd_ref.at[pl.ds(chunk * CHUNK, CHUNK), :],
            wd_buf.at[slot], wd_sem.at[slot]).start()

    @pl.when(c < NCHUNK)
    def _gu():
        x = x_ref[...]
        sl = pl.ds(c * CHUNK, CHUNK)
        g_s[:, sl] = jnp.dot(x, wg_ref[...], preferred_element_type=jnp.float32)
        u_s[:, sl] = jnp.dot(x, wu_ref[...], preferred_element_type=jnp.float32)

        @pl.when(c == NCHUNK - 1)
        def _prime():
            for k in range(NBUF):
                _start_wd(k, k)

    @pl.when(c == NCHUNK)
    def _threshold():
        keys = _keys_of(g_s[...])
        cnt_pos = jnp.sum((keys >= 0).astype(jnp.int32), axis=1,
                          keepdims=True)
        t0 = jnp.where(cnt_pos >= K, jnp.int32(0), jnp.int32(-(2**31)))

        def body(i, t):
            cand = t | (jnp.int32(1) << (jnp.int32(30) - i))
            cnt = jnp.sum((keys >= cand).astype(jnp.int32), axis=1,
                          keepdims=True)
            return jnp.where(cnt >= K, cand, t)

        t = lax.fori_loop(0, 31, body, t0)
        n_gt = jnp.sum((keys > t).astype(jnp.int32), axis=1, keepdims=True)
        col = lax.broadcasted_iota(jnp.int32, (m, 128), 1)
        tn_s[...] = jnp.where(col < 64, t, K - n_gt)
        carry_s[...] = jnp.zeros_like(carry_s)

    @pl.when(c > NCHUNK)
    def _down():
        cc = c - (NCHUNK + 1)
        g = g_s[:, pl.ds(cc * CHUNK, CHUNK)]
        u = u_s[:, pl.ds(cc * CHUNK, CHUNK)]
        kc = _keys_of(g)
        tn = tn_s[...]
        t = lax.slice(tn, (0, 0), (m, 1))
        need = lax.slice(tn, (0, 64), (m, 65)).astype(jnp.float32)
        ra = lax.broadcasted_iota(jnp.int32, (CHUNK, CHUNK), 0)
        rc = lax.broadcasted_iota(jnp.int32, (CHUNK, CHUNK), 1)
        tri = (ra <= rc).astype(jnp.float32)
        eq = (kc == t).astype(jnp.float32)
        carry = lax.slice(carry_s[...], (0, 0), (m, 1))
        cum = jnp.dot(eq, tri, preferred_element_type=jnp.float32) + carry
        sel = (kc > t) | ((kc == t) & (cum <= need))
        z = jnp.where(sel, g * (1.0 / (1.0 + jnp.exp(-g))) * u, 0.0)
        carry_s[...] = jnp.broadcast_to(
            carry + jnp.sum(eq, axis=1, keepdims=True), carry_s.shape)
        slot = lax.rem(cc, NBUF)
        pltpu.make_async_copy(
            wd_ref.at[pl.ds(cc * CHUNK, CHUNK), :],
            wd_buf.at[slot], wd_sem.at[slot]).wait()
        zd = jnp.dot(z, wd_buf[slot], preferred_element_type=jnp.float32)

        @pl.when(cc + NBUF < NCHUNK)
        def _refill():
            _start_wd(cc + NBUF, slot)

        @pl.when(cc == 0)
        def _first():
            acc_ref[...] = zd

        @pl.when(cc > 0)
        def _rest():
            acc_ref[...] += zd

        @pl.when(cc == NCHUNK - 1)
        def _emit():
            o_ref[...] = acc_ref[...]


def kernel(x, w_gate, w_up, w_down):
    orig_shape = x.shape
    xf = x.reshape(-1, orig_shape[-1])
    m = xf.shape[0]

    out = pl.pallas_call(
        _fused_kernel,
        grid=(2 * NCHUNK + 1,),
        in_specs=[
            pl.BlockSpec((m, D_MODEL), lambda c: (0, 0)),
            pl.BlockSpec((D_MODEL, CHUNK),
                         lambda c: (0, jnp.minimum(c, NCHUNK - 1))),
            pl.BlockSpec((D_MODEL, CHUNK),
                         lambda c: (0, jnp.minimum(c, NCHUNK - 1))),
            pl.BlockSpec(memory_space=pl.ANY),
        ],
        out_specs=pl.BlockSpec((m, D_MODEL), lambda c: (0, 0)),
        out_shape=jax.ShapeDtypeStruct((m, D_MODEL), jnp.float32),
        scratch_shapes=[
            pltpu.VMEM((m, D_FFN), jnp.float32),
            pltpu.VMEM((m, D_FFN), jnp.float32),
            pltpu.VMEM((m, 128), jnp.int32),
            pltpu.VMEM((m, 128), jnp.float32),
            pltpu.VMEM((m, D_MODEL), jnp.float32),
            pltpu.VMEM((NBUF, CHUNK, D_MODEL), jnp.float32),
            pltpu.SemaphoreType.DMA((NBUF,)),
        ],
    )(xf, w_gate, w_up, w_down)

    return out.reshape(orig_shape)
